# Initial kernel scaffold; baseline (speedup 1.0000x reference)
import jax
import jax.numpy as jnp
from jax.experimental import pallas as pl


def kernel(x, edge_index, W0, a_src0, a_dst0, b0, W1, a_src1, a_dst1, b1, W2, a_src2, a_dst2, b2):
    raise NotImplementedError("write your pallas kernel here")


# speedup vs baseline: 64.1722x; 64.1722x over previous
import functools

import jax
import jax.numpy as jnp
from jax import lax
from jax.experimental import pallas as pl
from jax.experimental.pallas import tpu as pltpu
from jax.experimental.pallas import tpu_sc as plsc

_N = 10000
_NP = 10240
_E = 320000
_C = 16
_NC, _NS = 2, 16
_NW = _NC * _NS
_EPT = _E // _NW
_B = 80
_NBLK = _EPT // _B
_RPT = _NP // _NS
_RBLK = 128
_R = 1024
_G = _NP // _R

_F32 = jnp.float32
_HIGH = lax.Precision.HIGHEST


def _dot(a, b):
    return jnp.dot(a, b, preferred_element_type=_F32, precision=_HIGH)


def _lrelu(x):
    return jnp.maximum(x, 0.2 * x)


def _emit_heads(i, h, ams, amd, as_ref, ad_ref, sh_ref, acc_ref, hn):
    asb = _dot(h, ams)
    adb = _dot(h, amd)
    as_ref[...] = asb
    ad_ref[...] = adb
    ms = jnp.max(asb, axis=0, keepdims=True)
    md = jnp.max(adb, axis=0, keepdims=True)

    @pl.when(i == 0)
    def _():
        acc_ref[0:1, :] = ms
        acc_ref[1:2, :] = md

    @pl.when(i > 0)
    def _():
        acc_ref[0:1, :] = jnp.maximum(acc_ref[0:1, :], ms)
        acc_ref[1:2, :] = jnp.maximum(acc_ref[1:2, :], md)

    s = jnp.maximum(acc_ref[0:1, 0:hn] + acc_ref[1:2, 0:hn], 0.0)
    sh_ref[...] = jnp.concatenate([s] * (16 // hn), axis=1)


def _pre0_body(x_ref, w_ref, ams_ref, amd_ref,
               h_ref, as_ref, ad_ref, sh_ref, acc_ref):
    i = pl.program_id(0)
    h = _dot(x_ref[...], w_ref[...])
    h_ref[...] = h
    _emit_heads(i, h, ams_ref[...], amd_ref[...], as_ref, ad_ref, sh_ref,
                acc_ref, 8)


def _pre0(x, w, ams, amd):
    return pl.pallas_call(
        _pre0_body,
        grid=(_G,),
        in_specs=[
            pl.BlockSpec((_R, 128), lambda i: (i, 0)),
            pl.BlockSpec((128, 128), lambda i: (0, 0)),
            pl.BlockSpec((128, 8), lambda i: (0, 0)),
            pl.BlockSpec((128, 8), lambda i: (0, 0)),
        ],
        out_specs=[
            pl.BlockSpec((_R, 128), lambda i: (i, 0)),
            pl.BlockSpec((_R, 8), lambda i: (i, 0)),
            pl.BlockSpec((_R, 8), lambda i: (i, 0)),
            pl.BlockSpec((1, 16), lambda i: (0, 0)),
        ],
        out_shape=[
            jax.ShapeDtypeStruct((_NP, 128), _F32),
            jax.ShapeDtypeStruct((_NP, 8), _F32),
            jax.ShapeDtypeStruct((_NP, 8), _F32),
            jax.ShapeDtypeStruct((1, 16), _F32),
        ],
        scratch_shapes=[pltpu.VMEM((2, 8), _F32)],
    )(x, w, ams, amd)


def _comb_body(o0_ref, o1_ref, d0_ref, d1_ref, h_ref, as_ref, ad_ref, sh_ref,
               b_ref, erep_ref, wn_ref, amsn_ref, amdn_ref,
               hn_ref, asn_ref, adn_ref, shn_ref, acc_ref, *, hn):
    i = pl.program_id(0)
    z = as_ref[...] + ad_ref[...]
    w = jnp.exp(_lrelu(z) - sh_ref[0:1, 0:8])
    num = o0_ref[...] + o1_ref[...] + _dot(w, erep_ref[...]) * h_ref[...]
    den = _dot(d0_ref[...] + d1_ref[...] + w, erep_ref[...])
    xin = jnp.maximum(num / den + b_ref[...], 0.0)
    hnv = _dot(xin, wn_ref[...])
    hn_ref[...] = hnv
    _emit_heads(i, hnv, amsn_ref[...], amdn_ref[...], asn_ref, adn_ref,
                shn_ref, acc_ref, hn)


def _combine(o0, o1, d0, d1, h, as_, ad_, sh, b, erep, wn, amsn, amdn, hn):
    dn = wn.shape[1]
    full = lambda r, c: pl.BlockSpec((r, c), lambda i: (0, 0))
    blk = lambda c: pl.BlockSpec((_R, c), lambda i: (i, 0))
    return pl.pallas_call(
        functools.partial(_comb_body, hn=hn),
        grid=(_G,),
        in_specs=[
            blk(128), blk(128), blk(8), blk(8), blk(128), blk(8), blk(8),
            full(1, 16), full(1, 128), full(8, 128),
            full(128, dn), full(dn, 8), full(dn, 8),
        ],
        out_specs=[blk(dn), blk(8), blk(8), full(1, 16)],
        out_shape=[
            jax.ShapeDtypeStruct((_NP, dn), _F32),
            jax.ShapeDtypeStruct((_NP, 8), _F32),
            jax.ShapeDtypeStruct((_NP, 8), _F32),
            jax.ShapeDtypeStruct((1, 16), _F32),
        ],
        scratch_shapes=[pltpu.VMEM((2, 8), _F32)],
    )(o0, o1, d0, d1, h, as_, ad_, sh, b, erep, wn, amsn, amdn)


def _final_body(o0_ref, o1_ref, d0_ref, d1_ref, h_ref, as_ref, ad_ref,
                sh_ref, b_ref, out_ref):
    z = as_ref[:, 0:1] + ad_ref[:, 0:1]
    w = jnp.exp(_lrelu(z) - sh_ref[0:1, 0:1])
    num = o0_ref[...] + o1_ref[...] + w * h_ref[...]
    den = d0_ref[:, 0:1] + d1_ref[:, 0:1] + w
    out_ref[...] = num / den + b_ref[...]


def _final(o0, o1, d0, d1, h, as_, ad_, sh, b):
    full = lambda r, c: pl.BlockSpec((r, c), lambda i: (0, 0))
    blk = lambda c: pl.BlockSpec((_R, c), lambda i: (i, 0))
    return pl.pallas_call(
        _final_body,
        grid=(_G,),
        in_specs=[blk(16), blk(16), blk(8), blk(8), blk(16), blk(8), blk(8),
                  full(1, 16), full(1, 16)],
        out_specs=blk(16),
        out_shape=jax.ShapeDtypeStruct((_NP, 16), _F32),
    )(o0, o1, d0, d1, h, as_, ad_, sh, b)


def _dyn_gather(vec, idx):
    return lax.gather(
        vec, idx[:, None],
        lax.GatherDimensionNumbers(offset_dims=(), collapsed_slice_dims=(0,),
                                   start_index_map=(0,)),
        slice_sizes=(1,), mode=lax.GatherScatterMode.PROMISE_IN_BOUNDS)


def _make_sc_edge(heads):
    d = _C * heads
    epv = 16 // heads
    nv = (_B * heads) // 16

    mesh = plsc.VectorSubcoreMesh(core_axis_name="c", subcore_axis_name="s",
                                  num_cores=_NC, num_subcores=_NS)
    out_type = (jax.ShapeDtypeStruct((2, _NP, d), _F32),
                jax.ShapeDtypeStruct((2, _NP, 8), _F32))
    scratch = [
        pltpu.VMEM((_B,), jnp.int32),
        pltpu.VMEM((_B,), jnp.int32),
        pltpu.VMEM((_B, 8), _F32),
        pltpu.VMEM((_B, 8), _F32),
        pltpu.VMEM((_B, 8), _F32),
        pltpu.VMEM((_B, d), _F32),
        pltpu.VMEM((16,), _F32),
        pltpu.VMEM((_RBLK, d), _F32),
        pltpu.VMEM((_RBLK, 8), _F32),
        pltpu.VMEM_SHARED((_NP, d), _F32),
        pltpu.VMEM_SHARED((_NP, 8), _F32),
        pltpu.SemaphoreType.DMA,
        pltpu.SemaphoreType.DMA,
        pltpu.SemaphoreType.DMA,
    ]

    @functools.partial(pl.kernel, out_type=out_type, mesh=mesh,
                       scratch_types=scratch,
                       compiler_params=pltpu.CompilerParams(
                           needs_layout_passes=False,
                           use_tc_tiling_on_sc=False))
    def edge_kernel(src_hbm, dst_hbm, h_hbm, as_hbm, ad_hbm, sh_hbm,
                    zd_hbm, z8_hbm, outp_hbm, denp_hbm,
                    sidx, didx, asg, adg, wbuf, rows, shv, fbuf, fden,
                    out_sp, den_sp, sem1, sem2, sem3):
        c = lax.axis_index("c")
        s = lax.axis_index("s")
        wid = c * _NS + s
        lane = lax.iota(jnp.int32, 16)

        pltpu.sync_copy(z8_hbm.at[pl.ds(0, _B)], wbuf)
        for chunk in range(_RPT // _RBLK):
            rbase = s * _RPT + chunk * _RBLK
            pltpu.sync_copy(zd_hbm, out_sp.at[pl.ds(rbase, _RBLK)])
            pltpu.sync_copy(z8_hbm, den_sp.at[pl.ds(rbase, _RBLK)])
        pltpu.sync_copy(sh_hbm, shv)
        plsc.subcore_barrier()

        tbase = wid * _EPT

        @pl.loop(0, _NBLK)
        def _block(j):
            base = tbase + j * _B
            pltpu.sync_copy(src_hbm.at[pl.ds(base, _B)], sidx)
            pltpu.sync_copy(dst_hbm.at[pl.ds(base, _B)], didx)
            cp1 = pltpu.async_copy(as_hbm.at[sidx], asg, sem1)
            cp2 = pltpu.async_copy(ad_hbm.at[didx], adg, sem2)
            cp3 = pltpu.async_copy(h_hbm.at[sidx], rows, sem3)
            cp1.wait()
            cp2.wait()
            sh = shv[...]

            @pl.loop(0, nv)
            def _wvec(v):
                r = lane // heads + epv * v
                cc = lane % heads
                z = (plsc.load_gather(asg, [r, cc])
                     + plsc.load_gather(adg, [r, cc]))
                w = jnp.exp(_lrelu(z) - sh)
                plsc.store_scatter(wbuf, [r, cc], w)

            cp3.wait()

            @pl.loop(0, _B)
            def _edge(e):
                er = lane * 0 + e
                w16 = plsc.load_gather(wbuf, [er, lane % heads])
                for hd in range(heads):
                    ws = _dyn_gather(w16, lane * 0 + hd) if heads > 1 else w16
                    sl = pl.ds(_C * hd, _C)
                    rows[e, sl] = rows[e, sl] * ws

            pltpu.sync_copy(wbuf, den_sp.at[didx], add=True)
            pltpu.sync_copy(rows, out_sp.at[didx], add=True)

        plsc.subcore_barrier()
        for chunk in range(_RPT // _RBLK):
            rbase = s * _RPT + chunk * _RBLK
            pltpu.sync_copy(out_sp.at[pl.ds(rbase, _RBLK)], fbuf)
            pltpu.sync_copy(fbuf, outp_hbm.at[c, pl.ds(rbase, _RBLK)])
            pltpu.sync_copy(den_sp.at[pl.ds(rbase, _RBLK)], fden)
            pltpu.sync_copy(fden, denp_hbm.at[c, pl.ds(rbase, _RBLK)])

    return edge_kernel


_sc_edge8 = _make_sc_edge(8)
_sc_edge1 = _make_sc_edge(1)


def _amats(a_src, a_dst, heads):
    eye = jnp.eye(heads, 8, dtype=_F32)
    ams = (a_src[:, :, None] * eye[:, None, :]).reshape(heads * _C, 8)
    amd = (a_dst[:, :, None] * eye[:, None, :]).reshape(heads * _C, 8)
    return ams, amd


def kernel(x, edge_index, W0, a_src0, a_dst0, b0, W1, a_src1, a_dst1, b1,
           W2, a_src2, a_dst2, b2):
    src = edge_index[0].astype(jnp.int32)
    dst = edge_index[1].astype(jnp.int32)
    x = jnp.pad(x, ((0, _NP - _N), (0, 0)))
    eye8 = jnp.eye(8, dtype=_F32)
    erep = (eye8[:, :, None] * jnp.ones((_C,), _F32)).reshape(8, 128)
    z128 = jnp.zeros((_RBLK, 128), _F32)
    z16 = jnp.zeros((_RBLK, 16), _F32)
    z8 = jnp.zeros((_RBLK, 8), _F32)

    ams0, amd0 = _amats(a_src0, a_dst0, 8)
    ams1, amd1 = _amats(a_src1, a_dst1, 8)
    ams2, amd2 = _amats(a_src2, a_dst2, 1)

    h0, as0, ad0, sh0 = _pre0(x, W0, ams0, amd0)
    op0, dp0 = _sc_edge8(src, dst, h0, as0, ad0, sh0.reshape(16), z128, z8)
    h1, as1, ad1, sh1 = _combine(op0[0], op0[1], dp0[0], dp0[1], h0, as0, ad0,
                                 sh0, b0.reshape(1, 128), erep, W1, ams1,
                                 amd1, 8)
    op1, dp1 = _sc_edge8(src, dst, h1, as1, ad1, sh1.reshape(16), z128, z8)
    h2, as2, ad2, sh2 = _combine(op1[0], op1[1], dp1[0], dp1[1], h1, as1, ad1,
                                 sh1, b1.reshape(1, 128), erep, W2, ams2,
                                 amd2, 1)
    op2, dp2 = _sc_edge1(src, dst, h2, as2, ad2, sh2.reshape(16), z16, z8)
    out = _final(op2[0], op2[1], dp2[0], dp2[1], h2, as2, ad2, sh2,
                 b2.reshape(1, 16))
    return out[:_N]


# speedup vs baseline: 91.3628x; 1.4237x over previous
import functools

import jax
import jax.numpy as jnp
from jax import lax
from jax.experimental import pallas as pl
from jax.experimental.pallas import tpu as pltpu
from jax.experimental.pallas import tpu_sc as plsc

_N = 10000
_NP = 10240
_E = 320000
_C = 16
_NC, _NS = 2, 16
_NW = _NC * _NS
_EPT = _E // _NW
_B = 80
_NBLK = _EPT // _B
_RPT = _NP // _NS
_RBLK = 128
_R = 1024
_G = _NP // _R

_F32 = jnp.float32
_HIGH = lax.Precision.HIGHEST


def _dot(a, b):
    return jnp.dot(a, b, preferred_element_type=_F32, precision=_HIGH)


def _lrelu(x):
    return jnp.maximum(x, 0.2 * x)


def _emit_heads(i, h, ams, amd, as_ref, ad_ref, sh_ref, acc_ref, hn):
    asb = _dot(h, ams)
    adb = _dot(h, amd)
    as_ref[...] = asb
    ad_ref[...] = adb
    ms = jnp.max(asb, axis=0, keepdims=True)
    md = jnp.max(adb, axis=0, keepdims=True)

    @pl.when(i == 0)
    def _():
        acc_ref[0:1, :] = ms
        acc_ref[1:2, :] = md

    @pl.when(i > 0)
    def _():
        acc_ref[0:1, :] = jnp.maximum(acc_ref[0:1, :], ms)
        acc_ref[1:2, :] = jnp.maximum(acc_ref[1:2, :], md)

    s = jnp.maximum(acc_ref[0:1, 0:hn] + acc_ref[1:2, 0:hn], 0.0)
    sh_ref[...] = jnp.concatenate([s] * (16 // hn), axis=1)


def _pre0_body(x_ref, w_ref, ams_ref, amd_ref,
               h_ref, as_ref, ad_ref, sh_ref, acc_ref):
    i = pl.program_id(0)
    h = _dot(x_ref[...], w_ref[...])
    h_ref[...] = h
    _emit_heads(i, h, ams_ref[...], amd_ref[...], as_ref, ad_ref, sh_ref,
                acc_ref, 8)


def _pre0(x, w, ams, amd):
    return pl.pallas_call(
        _pre0_body,
        grid=(_G,),
        in_specs=[
            pl.BlockSpec((_R, 128), lambda i: (i, 0)),
            pl.BlockSpec((128, 128), lambda i: (0, 0)),
            pl.BlockSpec((128, 8), lambda i: (0, 0)),
            pl.BlockSpec((128, 8), lambda i: (0, 0)),
        ],
        out_specs=[
            pl.BlockSpec((_R, 128), lambda i: (i, 0)),
            pl.BlockSpec((_R, 8), lambda i: (i, 0)),
            pl.BlockSpec((_R, 8), lambda i: (i, 0)),
            pl.BlockSpec((1, 16), lambda i: (0, 0)),
        ],
        out_shape=[
            jax.ShapeDtypeStruct((_NP, 128), _F32),
            jax.ShapeDtypeStruct((_NP, 8), _F32),
            jax.ShapeDtypeStruct((_NP, 8), _F32),
            jax.ShapeDtypeStruct((1, 16), _F32),
        ],
        scratch_shapes=[pltpu.VMEM((2, 8), _F32)],
    )(x, w, ams, amd)


def _comb_body(o0_ref, o1_ref, d0_ref, d1_ref, h_ref, as_ref, ad_ref,
               sh_ref, b_ref, erep_ref, wn_ref, amsn_ref, amdn_ref,
               hn_ref, asn_ref, adn_ref, shn_ref, acc_ref, *, hn):
    i = pl.program_id(0)
    z = as_ref[...] + ad_ref[...]
    w = jnp.exp(_lrelu(z) - sh_ref[0:1, 0:8])
    num = o0_ref[...] + o1_ref[...] + _dot(w, erep_ref[...]) * h_ref[...]
    den = _dot(d0_ref[...] + d1_ref[...] + w, erep_ref[...])
    xin = jnp.maximum(num / den + b_ref[...], 0.0)
    hnv = _dot(xin, wn_ref[...])
    hn_ref[...] = hnv
    _emit_heads(i, hnv, amsn_ref[...], amdn_ref[...], asn_ref, adn_ref,
                shn_ref, acc_ref, hn)


def _combine(o0, o1, d0, d1, h, as_, ad_, sh, b, erep, wn, amsn, amdn, hn):
    dn = wn.shape[1]
    full = lambda r, c: pl.BlockSpec((r, c), lambda i: (0, 0))
    blk = lambda c: pl.BlockSpec((_R, c), lambda i: (i, 0))
    return pl.pallas_call(
        functools.partial(_comb_body, hn=hn),
        grid=(_G,),
        in_specs=[
            blk(128), blk(128), blk(8), blk(8), blk(128), blk(8), blk(8),
            full(1, 16), full(1, 128), full(8, 128),
            full(128, dn), full(dn, 8), full(dn, 8),
        ],
        out_specs=[blk(dn), blk(8), blk(8), full(1, 16)],
        out_shape=[
            jax.ShapeDtypeStruct((_NP, dn), _F32),
            jax.ShapeDtypeStruct((_NP, 8), _F32),
            jax.ShapeDtypeStruct((_NP, 8), _F32),
            jax.ShapeDtypeStruct((1, 16), _F32),
        ],
        scratch_shapes=[pltpu.VMEM((2, 8), _F32)],
    )(o0, o1, d0, d1, h, as_, ad_, sh, b, erep, wn, amsn, amdn)


def _final_body(o0_ref, o1_ref, d0_ref, d1_ref, h_ref, as_ref, ad_ref,
                sh_ref, b_ref, out_ref):
    z = as_ref[:, 0:1] + ad_ref[:, 0:1]
    w = jnp.exp(_lrelu(z) - sh_ref[0:1, 0:1])
    num = o0_ref[...] + o1_ref[...] + w * h_ref[...]
    den = d0_ref[:, 0:1] + d1_ref[:, 0:1] + w
    out_ref[...] = num / den + b_ref[...]


def _final(o0, o1, d0, d1, h, as_, ad_, sh, b):
    full = lambda r, c: pl.BlockSpec((r, c), lambda i: (0, 0))
    blk = lambda c: pl.BlockSpec((_R, c), lambda i: (i, 0))
    return pl.pallas_call(
        _final_body,
        grid=(_G,),
        in_specs=[blk(16), blk(16), blk(8), blk(8), blk(16), blk(8), blk(8),
                  full(1, 16), full(1, 16)],
        out_specs=blk(16),
        out_shape=jax.ShapeDtypeStruct((_NP, 16), _F32),
    )(o0, o1, d0, d1, h, as_, ad_, sh, b)


def _dyn_gather(vec, idx):
    return lax.gather(
        vec, idx[:, None],
        lax.GatherDimensionNumbers(offset_dims=(), collapsed_slice_dims=(0,),
                                   start_index_map=(0,)),
        slice_sizes=(1,), mode=lax.GatherScatterMode.PROMISE_IN_BOUNDS)


def _make_sc_edge(heads):
    d = _C * heads
    epv = 16 // heads
    nv = (_B * heads) // 16

    mesh = plsc.VectorSubcoreMesh(core_axis_name="c", subcore_axis_name="s",
                                  num_cores=_NC, num_subcores=_NS)
    out_type = (jax.ShapeDtypeStruct((2, _NP, d), _F32),
                jax.ShapeDtypeStruct((2, _NP, 8), _F32))
    bufset = [
        pltpu.VMEM((_B,), jnp.int32),
        pltpu.VMEM((_B,), jnp.int32),
        pltpu.VMEM((_B, 8), _F32),
        pltpu.VMEM((_B, 8), _F32),
        pltpu.VMEM((_B, 8), _F32),
        pltpu.VMEM((_B, d), _F32),
    ]
    semset = [pltpu.SemaphoreType.DMA] * 7
    scratch = [
        *bufset, *bufset,
        pltpu.VMEM((16,), _F32),
        pltpu.VMEM((_RBLK, d), _F32),
        pltpu.VMEM((_RBLK, 8), _F32),
        pltpu.VMEM_SHARED((_NP, d), _F32),
        pltpu.VMEM_SHARED((_NP, 8), _F32),
    ] + semset + semset

    @functools.partial(pl.kernel, out_type=out_type, mesh=mesh,
                       scratch_types=scratch,
                       compiler_params=pltpu.CompilerParams(
                           needs_layout_passes=False,
                           use_tc_tiling_on_sc=False))
    def edge_kernel(src_hbm, dst_hbm, h_hbm, as_hbm, ad_hbm, sh_hbm,
                    zd_hbm, z8_hbm, outp_hbm, denp_hbm,
                    si0, di0, asg0, adg0, wb0, rows0,
                    si1, di1, asg1, adg1, wb1, rows1,
                    shv, fbuf, fden, out_sp, den_sp,
                    i10, i20, ga0, gb0, gc0, sw0, sr0,
                    i11, i21, ga1, gb1, gc1, sw1, sr1):
        c = lax.axis_index("c")
        s = lax.axis_index("s")
        wid = c * _NS + s
        lane = lax.iota(jnp.int32, 16)
        sets = ((si0, di0, asg0, adg0, wb0, rows0,
                 i10, i20, ga0, gb0, gc0, sw0, sr0),
                (si1, di1, asg1, adg1, wb1, rows1,
                 i11, i21, ga1, gb1, gc1, sw1, sr1))

        pltpu.sync_copy(z8_hbm.at[pl.ds(0, _B)], wb0)
        pltpu.sync_copy(z8_hbm.at[pl.ds(0, _B)], wb1)
        for chunk in range(_RPT // _RBLK):
            rbase = s * _RPT + chunk * _RBLK
            pltpu.sync_copy(zd_hbm, out_sp.at[pl.ds(rbase, _RBLK)])
            pltpu.sync_copy(z8_hbm, den_sp.at[pl.ds(rbase, _RBLK)])
        pltpu.sync_copy(sh_hbm, shv)
        plsc.subcore_barrier()

        tbase = wid * _EPT

        def issue(st, j):
            sidx, didx, asg, adg, wb, rows, i1, i2, ga, gb, gc, sw, sr = st
            base = tbase + j * _B
            a = pltpu.async_copy(src_hbm.at[pl.ds(base, _B)], sidx, i1)
            b = pltpu.async_copy(dst_hbm.at[pl.ds(base, _B)], didx, i2)
            a.wait()
            b.wait()
            return (pltpu.async_copy(as_hbm.at[sidx], asg, ga),
                    pltpu.async_copy(ad_hbm.at[didx], adg, gb),
                    pltpu.async_copy(h_hbm.at[sidx], rows, gc))

        def compute(st, cps):
            sidx, didx, asg, adg, wb, rows, *_ = st
            cps[0].wait()
            cps[1].wait()
            sh = shv[...]

            @pl.loop(0, nv)
            def _wvec(v):
                r = lane // heads + epv * v
                cc = lane % heads
                z = (plsc.load_gather(asg, [r, cc])
                     + plsc.load_gather(adg, [r, cc]))
                w = jnp.exp(_lrelu(z) - sh)
                plsc.store_scatter(wb, [r, cc], w)

            cps[2].wait()

            @pl.loop(0, _B)
            def _edge(e):
                er = lane * 0 + e
                w16 = plsc.load_gather(wb, [er, lane % heads])
                for hd in range(heads):
                    ws = _dyn_gather(w16, lane * 0 + hd) if heads > 1 else w16
                    sl = pl.ds(_C * hd, _C)
                    rows[e, sl] = rows[e, sl] * ws

        def scatter_start(st):
            sidx, didx, asg, adg, wb, rows, i1, i2, ga, gb, gc, sw, sr = st
            return (pltpu.async_copy(wb, den_sp.at[didx], sw, add=True),
                    pltpu.async_copy(rows, out_sp.at[didx], sr, add=True))

        @pl.loop(0, _NBLK // 2)
        def _pair(k):
            c0 = issue(sets[0], 2 * k)
            c1 = issue(sets[1], 2 * k + 1)
            compute(sets[0], c0)
            s0 = scatter_start(sets[0])
            compute(sets[1], c1)
            s1 = scatter_start(sets[1])
            s0[0].wait()
            s0[1].wait()
            s1[0].wait()
            s1[1].wait()

        compute(sets[0], issue(sets[0], _NBLK - 1))
        s0 = scatter_start(sets[0])
        s0[0].wait()
        s0[1].wait()

        plsc.subcore_barrier()
        for chunk in range(_RPT // _RBLK):
            rbase = s * _RPT + chunk * _RBLK
            pltpu.sync_copy(out_sp.at[pl.ds(rbase, _RBLK)], fbuf)
            pltpu.sync_copy(fbuf, outp_hbm.at[c, pl.ds(rbase, _RBLK)])
            pltpu.sync_copy(den_sp.at[pl.ds(rbase, _RBLK)], fden)
            pltpu.sync_copy(fden, denp_hbm.at[c, pl.ds(rbase, _RBLK)])

    return edge_kernel


_sc_edge8 = _make_sc_edge(8)
_sc_edge1 = _make_sc_edge(1)


def _amats(a_src, a_dst, heads):
    eye = jnp.eye(heads, 8, dtype=_F32)
    ams = (a_src[:, :, None] * eye[:, None, :]).reshape(heads * _C, 8)
    amd = (a_dst[:, :, None] * eye[:, None, :]).reshape(heads * _C, 8)
    return ams, amd


def kernel(x, edge_index, W0, a_src0, a_dst0, b0, W1, a_src1, a_dst1, b1,
           W2, a_src2, a_dst2, b2):
    src = edge_index[0].astype(jnp.int32)
    dst = edge_index[1].astype(jnp.int32)
    x = jnp.pad(x, ((0, _NP - _N), (0, 0)))
    eye8 = jnp.eye(8, dtype=_F32)
    erep = (eye8[:, :, None] * jnp.ones((_C,), _F32)).reshape(8, 128)
    z128 = jnp.zeros((_RBLK, 128), _F32)
    z16 = jnp.zeros((_RBLK, 16), _F32)
    z8 = jnp.zeros((_RBLK, 8), _F32)

    ams0, amd0 = _amats(a_src0, a_dst0, 8)
    ams1, amd1 = _amats(a_src1, a_dst1, 8)
    ams2, amd2 = _amats(a_src2, a_dst2, 1)

    h0, as0, ad0, sh0 = _pre0(x, W0, ams0, amd0)
    op0, dp0 = _sc_edge8(src, dst, h0, as0, ad0, sh0.reshape(16), z128, z8)
    h1, as1, ad1, sh1 = _combine(op0[0], op0[1], dp0[0], dp0[1], h0, as0, ad0,
                                 sh0, b0.reshape(1, 128), erep, W1, ams1,
                                 amd1, 8)
    op1, dp1 = _sc_edge8(src, dst, h1, as1, ad1, sh1.reshape(16), z128, z8)
    h2, as2, ad2, sh2 = _combine(op1[0], op1[1], dp1[0], dp1[1], h1, as1, ad1,
                                 sh1, b1.reshape(1, 128), erep, W2, ams2,
                                 amd2, 1)
    op2, dp2 = _sc_edge1(src, dst, h2, as2, ad2, sh2.reshape(16), z16, z8)
    out = _final(op2[0], op2[1], dp2[0], dp2[1], h2, as2, ad2, sh2,
                 b2.reshape(1, 16))
    return out[:_N]
